# baseline (speedup 1.0000x reference)
import functools

import jax
import jax.numpy as jnp
from jax import lax
from jax.experimental import pallas as pl
from jax.experimental.pallas import tpu as pltpu
from jax.experimental.pallas import tpu_sc as plsc

_DIM = 32
_BATCH = 16384
_NC = 2
_NS = 16
_L = 16
_NW = _NC * _NS
_BPW = _BATCH // _NW
_CHUNK = 128
_NCHUNKS = _BPW // _CHUNK


def _sc_kernel(table_hbm, idx_hbm, out_hbm, idx_v, rows_v, out_v, sems):
  wid = lax.axis_index("s") * _NC + lax.axis_index("c")

  pltpu.sync_copy(idx_hbm.at[wid], idx_v)

  copies = []
  for j in range(_NCHUNKS):
    copies.append(
        pltpu.async_copy(
            table_hbm.at[idx_v.at[j]],
            rows_v.at[pl.ds(j * _CHUNK, _CHUNK)],
            sems.at[j],
        )
    )

  even = lax.iota(jnp.int32, _L) * 2
  odd = even + 1

  for j in range(_NCHUNKS):
    copies[j].wait()

    def convert_row(r, _):
      base = r * _DIM
      row = jnp.full((_L,), r, jnp.int32)
      a = plsc.load_gather(rows_v, [row, even])
      b = plsc.load_gather(rows_v, [row, odd])
      packed = plsc.pack(a, b, format=plsc.PackFormat.INTERLEAVED)
      out_v[pl.ds(base, _DIM)] = packed
      return 0

    lax.fori_loop(j * _CHUNK, (j + 1) * _CHUNK, convert_row, 0)

  pltpu.sync_copy(out_v, out_hbm.at[pl.ds(wid * _BPW * _DIM, _BPW * _DIM)])


@jax.jit
def _lookup(table, idx):
  mesh = plsc.VectorSubcoreMesh(core_axis_name="c", subcore_axis_name="s")
  f = pl.kernel(
      _sc_kernel,
      out_type=jax.ShapeDtypeStruct((_BATCH * _DIM,), jnp.bfloat16),
      mesh=mesh,
      scratch_types=[
          pltpu.VMEM((_NCHUNKS, _CHUNK), jnp.int32),
          pltpu.VMEM((_BPW, _DIM), jnp.float32),
          pltpu.VMEM((_BPW * _DIM,), jnp.bfloat16),
          pltpu.SemaphoreType.DMA((_NCHUNKS,)),
      ],
      compiler_params=pltpu.CompilerParams(
          needs_layout_passes=False, use_tc_tiling_on_sc=False
      ),
  )
  return f(table, idx)


def kernel(inputs, weights):
  idx = inputs.astype(jnp.int32).reshape(_NW, _NCHUNKS, _CHUNK)
  out = _lookup(weights, idx)
  return out.reshape(_BATCH, _DIM)
